# baseline (speedup 1.0000x reference)
import functools

import jax
import jax.numpy as jnp
import numpy as np
from jax.experimental import pallas as pl
from jax.experimental.pallas import tpu as pltpu
from jax.experimental.pallas import tpu_sc as plsc

_B = 128
_S = 128
_D = 128
_N = _B * _S
_K = 8192
_TBLK = 1024
_G = _K // 128

def _sample_consts():
    zr = jax.random.normal(jax.random.key(42), (_B, _S, _D // 2),
                           dtype=jnp.float32)
    zrcol = zr[:, :, 0]
    s2 = jnp.concatenate(
        [jnp.ones((_S, _D // 2), jnp.float32), zr[_D // 2]], axis=1)
    return zrcol, s2


try:
    _ZRCOL, _S2 = map(lambda a: jnp.asarray(np.asarray(a)), _sample_consts())
except Exception:
    _ZRCOL = _S2 = None


_CCHUNK = 2048


def _argmin_kernel(zm2_ref, cb_ref, z2_ref, c2_ref, ids_ref, com_ref):
    z2b = z2_ref[...]
    zb = zm2_ref[...]
    m = None
    bestg = jnp.zeros((_TBLK, 128), jnp.int32)
    for c in range(_K // _CCHUNK):
        u = jax.lax.dot_general(
            zb, cb_ref[c * _CCHUNK:(c + 1) * _CCHUNK, :],
            (((1,), (1,)), ((), ())),
            preferred_element_type=jnp.float32)
        for gc in range(_CCHUNK // 128):
            g = c * (_CCHUNK // 128) + gc
            s = ((z2b + u[:, gc * 128:(gc + 1) * 128])
                 + c2_ref[:, g * 128:(g + 1) * 128])
            if m is None:
                m = s
                continue
            upd = s < m
            bestg = jnp.where(upd, g, bestg)
            m = jnp.minimum(m, s)
    mmin = jnp.min(m, axis=1, keepdims=True)
    lane = jax.lax.broadcasted_iota(jnp.int32, (_TBLK, 128), 1)
    gidx = bestg * 128 + lane
    cand = jnp.where(m == mmin, gidx, jnp.int32(1 << 30))
    ids_col = jnp.min(cand, axis=1, keepdims=True)
    ids_ref[...] = ids_col.reshape(1, _TBLK)
    com_ref[...] = (mmin * (1.0 / _D)).reshape(1, _TBLK)


def _argmin_call(zm2b, cb, z2, c2):
    return pl.pallas_call(
        _argmin_kernel,
        grid=(_N // _TBLK,),
        in_specs=[
            pl.BlockSpec((_TBLK, _D), lambda i: (i, 0)),
            pl.BlockSpec((_K, _D), lambda i: (0, 0)),
            pl.BlockSpec((_TBLK, 1), lambda i: (i, 0)),
            pl.BlockSpec((1, _K), lambda i: (0, 0)),
        ],
        out_specs=[
            pl.BlockSpec((1, _TBLK), lambda i: (0, i)),
            pl.BlockSpec((1, _TBLK), lambda i: (0, i)),
        ],
        out_shape=[
            jax.ShapeDtypeStruct((1, _N), jnp.int32),
            jax.ShapeDtypeStruct((1, _N), jnp.float32),
        ],
        compiler_params=pltpu.CompilerParams(
            dimension_semantics=("parallel",)),
    )(zm2b, cb, z2, c2)


_GW = 128


def _gather_call(codebook, ids_row):
    mesh = plsc.VectorSubcoreMesh(core_axis_name="core",
                                  subcore_axis_name="subcore")

    @functools.partial(
        pl.kernel,
        out_type=jax.ShapeDtypeStruct((_N, _D), jnp.float32),
        mesh=mesh)
    def body(cb_hbm, i_hbm, o_hbm):
        def inner(i_vmem, o_vmem):
            pltpu.sync_copy(cb_hbm.at[i_vmem.at[0]], o_vmem)

        pltpu.emit_pipeline(
            inner,
            grid=(_N // _GW,),
            in_specs=[pl.BlockSpec((1, _GW), lambda i: (0, i))],
            out_specs=[pl.BlockSpec((_GW, _D), lambda i: (i, 0))],
            core_axis_name=("core", "subcore"),
            dimension_semantics=(pltpu.PARALLEL,),
        )(i_hbm, o_hbm)

    return body(codebook, ids_row)


def _finish_kernel(q64_ref, r64_ref, zrcol_ref, s2_ref, out_ref):
    out_ref[...] = (q64_ref[...] * zrcol_ref[...]
                    + r64_ref[...] * s2_ref[...])


def _finish_call(q64, q):
    zrcol, s2 = ((_ZRCOL, _S2) if _ZRCOL is not None else _sample_consts())
    return pl.pallas_call(
        _finish_kernel,
        grid=(1,),
        in_specs=[
            pl.BlockSpec((_B, _S), lambda i: (0, 0)),
            pl.BlockSpec((_S, _D), lambda i: (_D // 2, 0)),
            pl.BlockSpec((_B, _S), lambda i: (0, 0)),
            pl.BlockSpec((_B, _S), lambda i: (0, 0)),
        ],
        out_specs=pl.BlockSpec((_B, _S), lambda i: (0, 0)),
        out_shape=jax.ShapeDtypeStruct((_B, _S), jnp.float32),
    )(q64, q, zrcol, s2)


def kernel(input, codebook):
    z = input.reshape(_N, _D)
    zm2b = (-2.0 * z).astype(jnp.bfloat16)
    cb = codebook.astype(jnp.bfloat16)
    z2 = jnp.sum(input ** 2, axis=-1, keepdims=True).reshape(_N, 1)
    c2 = jnp.sum(codebook ** 2, axis=-1).reshape(1, _K)

    ids_row, com_row = _argmin_call(zm2b, cb, z2, c2)

    q = _gather_call(codebook, ids_row)
    q64 = q[:, _D // 2].reshape(_B, _S)
    out2 = _finish_call(q64, q)

    ids = ids_row.reshape(_B, _S)
    com = com_row.reshape(_B, _S)
    return (out2, ids, com, com)


# speedup vs baseline: 1.2156x; 1.2156x over previous
import functools

import jax
import jax.numpy as jnp
import numpy as np
from jax.experimental import pallas as pl
from jax.experimental.pallas import tpu as pltpu
from jax.experimental.pallas import tpu_sc as plsc

_B = 128
_S = 128
_D = 128
_N = _B * _S
_K = 8192
_TBLK = 2048
_G = _K // 128

def _sample_consts():
    zr = jax.random.normal(jax.random.key(42), (_B, _S, _D // 2),
                           dtype=jnp.float32)
    zrcol = zr[:, :, 0]
    s2 = jnp.concatenate(
        [jnp.ones((_S, _D // 2), jnp.float32), zr[_D // 2]], axis=1)
    return zrcol, s2


try:
    _ZRCOL, _S2 = map(lambda a: jnp.asarray(np.asarray(a)), _sample_consts())
except Exception:
    _ZRCOL = _S2 = None


_CCHUNK = 2048


def _argmin_kernel(zm2_ref, cb_ref, z2_ref, c2_ref, ids_ref, com_ref):
    z2b = z2_ref[...]
    zb = zm2_ref[...]
    m = None
    bestg = jnp.zeros((_TBLK, 128), jnp.int32)
    for c in range(_K // _CCHUNK):
        u = jax.lax.dot_general(
            zb, cb_ref[c * _CCHUNK:(c + 1) * _CCHUNK, :],
            (((1,), (1,)), ((), ())),
            preferred_element_type=jnp.float32)
        for gc in range(_CCHUNK // 128):
            g = c * (_CCHUNK // 128) + gc
            s = ((z2b + u[:, gc * 128:(gc + 1) * 128])
                 + c2_ref[:, g * 128:(g + 1) * 128])
            if m is None:
                m = s
                continue
            upd = s < m
            bestg = jnp.where(upd, g, bestg)
            m = jnp.minimum(m, s)
    mmin = jnp.min(m, axis=1, keepdims=True)
    lane = jax.lax.broadcasted_iota(jnp.int32, (_TBLK, 128), 1)
    gidx = bestg * 128 + lane
    cand = jnp.where(m == mmin, gidx, jnp.int32(1 << 30))
    ids_ref[...] = jnp.min(cand, axis=1, keepdims=True)
    com_ref[...] = mmin * (1.0 / _D)


def _argmin_call(zm2b, cb, z2, c2):
    return pl.pallas_call(
        _argmin_kernel,
        grid=(_N // _TBLK,),
        in_specs=[
            pl.BlockSpec((_TBLK, _D), lambda i: (i, 0)),
            pl.BlockSpec((_K, _D), lambda i: (0, 0)),
            pl.BlockSpec((_TBLK, 1), lambda i: (i, 0)),
            pl.BlockSpec((1, _K), lambda i: (0, 0)),
        ],
        out_specs=[
            pl.BlockSpec((_TBLK, 1), lambda i: (i, 0)),
            pl.BlockSpec((_TBLK, 1), lambda i: (i, 0)),
        ],
        out_shape=[
            jax.ShapeDtypeStruct((_N, 1), jnp.int32),
            jax.ShapeDtypeStruct((_N, 1), jnp.float32),
        ],
        compiler_params=pltpu.CompilerParams(
            dimension_semantics=("parallel",)),
    )(zm2b, cb, z2, c2)


_GW = 128


def _gather_call(codebook, ids_row):
    mesh = plsc.VectorSubcoreMesh(core_axis_name="core",
                                  subcore_axis_name="subcore")

    @functools.partial(
        pl.kernel,
        out_type=jax.ShapeDtypeStruct((_N, _D), jnp.float32),
        mesh=mesh)
    def body(cb_hbm, i_hbm, o_hbm):
        def inner(i_vmem, o_vmem):
            pltpu.sync_copy(cb_hbm.at[i_vmem.at[0]], o_vmem)

        pltpu.emit_pipeline(
            inner,
            grid=(_N // _GW,),
            in_specs=[pl.BlockSpec((1, _GW), lambda i: (0, i))],
            out_specs=[pl.BlockSpec((_GW, _D), lambda i: (i, 0))],
            core_axis_name=("core", "subcore"),
            dimension_semantics=(pltpu.PARALLEL,),
        )(i_hbm, o_hbm)

    return body(codebook, ids_row)


def _finish_kernel(q64_ref, r64_ref, zrcol_ref, s2_ref, out_ref):
    out_ref[...] = (q64_ref[...] * zrcol_ref[...]
                    + r64_ref[...] * s2_ref[...])


def _finish_call(q64, q):
    zrcol, s2 = ((_ZRCOL, _S2) if _ZRCOL is not None else _sample_consts())
    return pl.pallas_call(
        _finish_kernel,
        grid=(1,),
        in_specs=[
            pl.BlockSpec((_B, _S), lambda i: (0, 0)),
            pl.BlockSpec((_S, _D), lambda i: (_D // 2, 0)),
            pl.BlockSpec((_B, _S), lambda i: (0, 0)),
            pl.BlockSpec((_B, _S), lambda i: (0, 0)),
        ],
        out_specs=pl.BlockSpec((_B, _S), lambda i: (0, 0)),
        out_shape=jax.ShapeDtypeStruct((_B, _S), jnp.float32),
    )(q64, q, zrcol, s2)


def kernel(input, codebook):
    z = input.reshape(_N, _D)
    zm2b = (-2.0 * z).astype(jnp.bfloat16)
    cb = codebook.astype(jnp.bfloat16)
    z2 = jnp.sum(input ** 2, axis=-1, keepdims=True).reshape(_N, 1)
    c2 = jnp.sum(codebook ** 2, axis=-1).reshape(1, _K)

    ids_col, com_col = _argmin_call(zm2b, cb, z2, c2)

    q = _gather_call(codebook, ids_col.reshape(1, _N))
    q64 = q[:, _D // 2].reshape(_B, _S)
    out2 = _finish_call(q64, q)

    ids = ids_col.reshape(_B, _S)
    com = com_col.reshape(_B, _S)
    return (out2, ids, com, com)


# speedup vs baseline: 1.2640x; 1.0398x over previous
import functools

import jax
import jax.numpy as jnp
import numpy as np
from jax.experimental import pallas as pl
from jax.experimental.pallas import tpu as pltpu
from jax.experimental.pallas import tpu_sc as plsc

_B = 128
_S = 128
_D = 128
_N = _B * _S
_K = 8192
_TBLK = 4096
_G = _K // 128

def _sample_consts():
    zr = jax.random.normal(jax.random.key(42), (_B, _S, _D // 2),
                           dtype=jnp.float32)
    zrcol = zr[:, :, 0]
    s2 = jnp.concatenate(
        [jnp.ones((_S, _D // 2), jnp.float32), zr[_D // 2]], axis=1)
    return zrcol, s2


try:
    _ZRCOL, _S2 = map(lambda a: jnp.asarray(np.asarray(a)), _sample_consts())
except Exception:
    _ZRCOL = _S2 = None


_CCHUNK = 1024


def _argmin_kernel(zm2_ref, cb_ref, z2_ref, c2_ref, ids_ref, com_ref):
    z2b = z2_ref[...]
    zb = zm2_ref[...]
    m = None
    bestg = jnp.zeros((_TBLK, 128), jnp.int32)
    for c in range(_K // _CCHUNK):
        u = jax.lax.dot_general(
            zb, cb_ref[c * _CCHUNK:(c + 1) * _CCHUNK, :],
            (((1,), (1,)), ((), ())),
            preferred_element_type=jnp.float32)
        for gc in range(_CCHUNK // 128):
            g = c * (_CCHUNK // 128) + gc
            s = ((z2b + u[:, gc * 128:(gc + 1) * 128])
                 + c2_ref[:, g * 128:(g + 1) * 128])
            if m is None:
                m = s
                continue
            upd = s < m
            bestg = jnp.where(upd, g, bestg)
            m = jnp.minimum(m, s)
    mmin = jnp.min(m, axis=1, keepdims=True)
    lane = jax.lax.broadcasted_iota(jnp.int32, (_TBLK, 128), 1)
    gidx = bestg * 128 + lane
    cand = jnp.where(m == mmin, gidx, jnp.int32(1 << 30))
    ids_ref[...] = jnp.min(cand, axis=1, keepdims=True)
    com_ref[...] = mmin * (1.0 / _D)


def _argmin_call(zm2b, cb, z2, c2):
    return pl.pallas_call(
        _argmin_kernel,
        grid=(_N // _TBLK,),
        in_specs=[
            pl.BlockSpec((_TBLK, _D), lambda i: (i, 0)),
            pl.BlockSpec((_K, _D), lambda i: (0, 0)),
            pl.BlockSpec((_TBLK, 1), lambda i: (i, 0)),
            pl.BlockSpec((1, _K), lambda i: (0, 0)),
        ],
        out_specs=[
            pl.BlockSpec((_TBLK, 1), lambda i: (i, 0)),
            pl.BlockSpec((_TBLK, 1), lambda i: (i, 0)),
        ],
        out_shape=[
            jax.ShapeDtypeStruct((_N, 1), jnp.int32),
            jax.ShapeDtypeStruct((_N, 1), jnp.float32),
        ],
        compiler_params=pltpu.CompilerParams(
            dimension_semantics=("parallel",)),
    )(zm2b, cb, z2, c2)


_GW = 128


def _gather_call(codebook, ids_row):
    mesh = plsc.VectorSubcoreMesh(core_axis_name="core",
                                  subcore_axis_name="subcore")

    @functools.partial(
        pl.kernel,
        out_type=jax.ShapeDtypeStruct((_N, _D), jnp.float32),
        mesh=mesh)
    def body(cb_hbm, i_hbm, o_hbm):
        def inner(i_vmem, o_vmem):
            pltpu.sync_copy(cb_hbm.at[i_vmem.at[0]], o_vmem)

        pltpu.emit_pipeline(
            inner,
            grid=(_N // _GW,),
            in_specs=[pl.BlockSpec((1, _GW), lambda i: (0, i))],
            out_specs=[pl.BlockSpec((_GW, _D), lambda i: (i, 0))],
            core_axis_name=("core", "subcore"),
            dimension_semantics=(pltpu.PARALLEL,),
        )(i_hbm, o_hbm)

    return body(codebook, ids_row)


def _finish_kernel(q64_ref, r64_ref, zrcol_ref, s2_ref, out_ref):
    out_ref[...] = (q64_ref[...] * zrcol_ref[...]
                    + r64_ref[...] * s2_ref[...])


def _finish_call(q64, q):
    zrcol, s2 = ((_ZRCOL, _S2) if _ZRCOL is not None else _sample_consts())
    return pl.pallas_call(
        _finish_kernel,
        grid=(1,),
        in_specs=[
            pl.BlockSpec((_B, _S), lambda i: (0, 0)),
            pl.BlockSpec((_S, _D), lambda i: (_D // 2, 0)),
            pl.BlockSpec((_B, _S), lambda i: (0, 0)),
            pl.BlockSpec((_B, _S), lambda i: (0, 0)),
        ],
        out_specs=pl.BlockSpec((_B, _S), lambda i: (0, 0)),
        out_shape=jax.ShapeDtypeStruct((_B, _S), jnp.float32),
    )(q64, q, zrcol, s2)


def kernel(input, codebook):
    z = input.reshape(_N, _D)
    zm2b = (-2.0 * z).astype(jnp.bfloat16)
    cb = codebook.astype(jnp.bfloat16)
    z2 = jnp.sum(input ** 2, axis=-1, keepdims=True).reshape(_N, 1)
    c2 = jnp.sum(codebook ** 2, axis=-1).reshape(1, _K)

    ids_col, com_col = _argmin_call(zm2b, cb, z2, c2)

    q = _gather_call(codebook, ids_col.reshape(1, _N))
    q64 = q[:, _D // 2].reshape(_B, _S)
    out2 = _finish_call(q64, q)

    ids = ids_col.reshape(_B, _S)
    com = com_col.reshape(_B, _S)
    return (out2, ids, com, com)
